# baseline (speedup 1.0000x reference)
import functools

import jax
import jax.numpy as jnp
from jax import lax
from jax.experimental import pallas as pl
from jax.experimental.pallas import tpu as pltpu
from jax.experimental.pallas import tpu_sc as plsc

B = 4
C = 192
HW = 384 * 384
CHW = C * HW
NUM_P = 256
ELEMS = B * NUM_P * C

_info = plsc.get_sparse_core_info()
NC, NS, L = _info.num_cores, _info.num_subcores, _info.num_lanes
NW = NC * NS
ELEMS_PER_W = ELEMS // NW
IDX_MINOR = 128
IDX_MAJOR = ELEMS_PER_W // IDX_MINOR
OUT_ROWS = ELEMS // IDX_MINOR
C8_PER_W = 3


@functools.partial(
    pl.kernel,
    out_type=(
        jax.ShapeDtypeStruct((ELEMS,), jnp.float32),
        jax.ShapeDtypeStruct((B * NUM_P,), jnp.int32),
    ),
    mesh=plsc.VectorSubcoreMesh(core_axis_name="c", subcore_axis_name="s"),
    scratch_types=[
        pltpu.VMEM((NUM_P,), jnp.int32),
        pltpu.VMEM((ELEMS_PER_W,), jnp.int32),
        pltpu.VMEM((ELEMS_PER_W,), jnp.float32),
        pltpu.SemaphoreType.DMA,
    ],
)
def _sc_gather(feats_hbm, pids_hbm, out_hbm, ids_hbm, fpid_v, idx_v, rows_v, sem):
    t = lax.axis_index("c") * NS + lax.axis_index("s")
    b = lax.shift_right_logical(t, 3)
    cid = lax.axis_index("c")
    half = B * NUM_P // NC

    @pl.when(lax.axis_index("s") == 0)
    def _():
        pltpu.sync_copy(
            pids_hbm.at[pl.ds(pl.multiple_of(cid * half, half), half)],
            ids_hbm.at[pl.ds(pl.multiple_of(cid * half, half), half)],
        )
    for p128 in range(NUM_P // IDX_MINOR):
        pltpu.sync_copy(
            pids_hbm.at[pl.ds((p128 * B + b) * IDX_MINOR, IDX_MINOR)],
            fpid_v.at[pl.ds(p128 * IDX_MINOR, IDX_MINOR)],
        )

    def pid_fn(k, carry):
        p = fpid_v[pl.ds(k * L, L)]
        q = lax.shift_right_logical(p, 7)
        rem = lax.bitwise_and(p, 127)
        h = lax.shift_right_logical(q * 43691, 17)
        wq = q - 3 * h
        fpid_v[pl.ds(k * L, L)] = (
            lax.shift_right_logical(h, 3) * 3072
            + wq * 1024
            + lax.bitwise_and(h, 7) * 128
            + rem
        )
        return carry

    lax.fori_loop(0, NUM_P // L, pid_fn, 0)

    base_b = b * CHW

    def row_fn(j, carry):
        c8r = lax.shift_right_logical(j, 4)
        p128 = lax.bitwise_and(lax.shift_right_logical(j, 3), 1)
        cm8 = lax.bitwise_and(j, 7)
        c = ((lax.bitwise_and(t, 7) * C8_PER_W + c8r) * 8) + cm8
        base = base_b + c * HW
        for kk in range(IDX_MINOR // L):
            idx_v[pl.ds(j * IDX_MINOR + kk * L, L)] = (
                fpid_v[pl.ds(p128 * IDX_MINOR + kk * L, L)] + base
            )
        return carry

    lax.fori_loop(0, IDX_MAJOR, row_fn, 0)

    pltpu.async_copy(feats_hbm.at[idx_v], rows_v, sem).wait()
    pltpu.sync_copy(
        rows_v,
        out_hbm.at[pl.ds(pl.multiple_of(t * ELEMS_PER_W, ELEMS_PER_W), ELEMS_PER_W)],
    )


def _norm_body(x_ref, o_ref):
    x = x_ref[...].reshape(B, C // 8, NUM_P // 128, 8, 128)
    s = jnp.sum(x * x, axis=(1, 3), keepdims=True)
    o = x / (jnp.sqrt(s) + 1e-7)
    o_ref[...] = o.reshape(OUT_ROWS, IDX_MINOR)


def _normalize(x):
    return pl.pallas_call(
        _norm_body,
        out_shape=jax.ShapeDtypeStruct((OUT_ROWS, IDX_MINOR), jnp.float32),
    )(x)


def kernel(feats, num_patches, patch_ids):
    del num_patches
    feats_flat = (
        feats.reshape(B, C, 48, 8, 3, 128)
        .transpose(0, 1, 2, 4, 3, 5)
        .reshape(-1)
    )
    pids_flat = (
        patch_ids.reshape(B, NUM_P // 128, 128)
        .transpose(1, 0, 2)
        .reshape(-1)
    )
    gathered, ids_phys = _sc_gather(feats_flat, pids_flat)
    normed = _normalize(gathered.reshape(OUT_ROWS, IDX_MINOR))
    out = (
        normed.reshape(B, C // 8, NUM_P // 128, 8, 128)
        .transpose(0, 2, 4, 1, 3)
        .reshape(B, NUM_P, C)
    )
    ids = (
        ids_phys.reshape(NUM_P // 128, B, 128)
        .transpose(1, 0, 2)
        .reshape(B, NUM_P)
    )
    return out, ids


# speedup vs baseline: 1.0173x; 1.0173x over previous
import functools

import jax
import jax.numpy as jnp
from jax import lax
from jax.experimental import pallas as pl
from jax.experimental.pallas import tpu as pltpu
from jax.experimental.pallas import tpu_sc as plsc

B = 4
C = 192
HW = 384 * 384
CHW = C * HW
NUM_P = 256
ELEMS = B * NUM_P * C

_info = plsc.get_sparse_core_info()
NC, NS, L = _info.num_cores, _info.num_subcores, _info.num_lanes
NW = NC * NS
ELEMS_PER_W = ELEMS // NW
IDX_MINOR = 128
IDX_MAJOR = ELEMS_PER_W // IDX_MINOR
OUT_ROWS = ELEMS // IDX_MINOR
C8_PER_W = 3


@functools.partial(
    pl.kernel,
    out_type=jax.ShapeDtypeStruct((ELEMS,), jnp.float32),
    mesh=plsc.VectorSubcoreMesh(core_axis_name="c", subcore_axis_name="s"),
    scratch_types=[
        pltpu.VMEM((NUM_P,), jnp.int32),
        pltpu.VMEM((ELEMS_PER_W,), jnp.int32),
        pltpu.VMEM((ELEMS_PER_W,), jnp.float32),
        pltpu.SemaphoreType.DMA,
    ],
)
def _sc_gather(feats_hbm, pids_hbm, out_hbm, fpid_v, idx_v, rows_v, sem):
    t = lax.axis_index("c") * NS + lax.axis_index("s")
    b = lax.shift_right_logical(t, 3)
    for p128 in range(NUM_P // IDX_MINOR):
        pltpu.sync_copy(
            pids_hbm.at[pl.ds((p128 * B + b) * IDX_MINOR, IDX_MINOR)],
            fpid_v.at[pl.ds(p128 * IDX_MINOR, IDX_MINOR)],
        )

    def pid_fn(k, carry):
        p = fpid_v[pl.ds(k * L, L)]
        q = lax.shift_right_logical(p, 7)
        rem = lax.bitwise_and(p, 127)
        h = lax.shift_right_logical(q * 43691, 17)
        wq = q - 3 * h
        fpid_v[pl.ds(k * L, L)] = (
            lax.shift_right_logical(h, 3) * 3072
            + wq * 1024
            + lax.bitwise_and(h, 7) * 128
            + rem
        )
        return carry

    lax.fori_loop(0, NUM_P // L, pid_fn, 0)

    base_b = b * CHW

    def row_fn(j, carry):
        c8r = lax.shift_right_logical(j, 4)
        p128 = lax.bitwise_and(lax.shift_right_logical(j, 3), 1)
        cm8 = lax.bitwise_and(j, 7)
        c = ((lax.bitwise_and(t, 7) * C8_PER_W + c8r) * 8) + cm8
        base = base_b + c * HW
        for kk in range(IDX_MINOR // L):
            idx_v[pl.ds(j * IDX_MINOR + kk * L, L)] = (
                fpid_v[pl.ds(p128 * IDX_MINOR + kk * L, L)] + base
            )
        return carry

    lax.fori_loop(0, IDX_MAJOR, row_fn, 0)

    pltpu.async_copy(feats_hbm.at[idx_v], rows_v, sem).wait()
    pltpu.sync_copy(
        rows_v,
        out_hbm.at[pl.ds(pl.multiple_of(t * ELEMS_PER_W, ELEMS_PER_W), ELEMS_PER_W)],
    )


def _norm_body(x_ref, o_ref):
    x = x_ref[...].reshape(B, C // 8, NUM_P // 128, 8, 128)
    s = jnp.sum(x * x, axis=(1, 3), keepdims=True)
    o = x / (jnp.sqrt(s) + 1e-7)
    o_ref[...] = o.reshape(OUT_ROWS, IDX_MINOR)


def _normalize(x):
    return pl.pallas_call(
        _norm_body,
        out_shape=jax.ShapeDtypeStruct((OUT_ROWS, IDX_MINOR), jnp.float32),
    )(x)


def kernel(feats, num_patches, patch_ids):
    del num_patches
    feats_flat = (
        feats.reshape(B, C, 48, 8, 3, 128)
        .transpose(0, 1, 2, 4, 3, 5)
        .reshape(-1)
    )
    pids_flat = (
        patch_ids.reshape(B, NUM_P // 128, 128)
        .transpose(1, 0, 2)
        .reshape(-1)
    )
    gathered = _sc_gather(feats_flat, pids_flat)
    normed = _normalize(gathered.reshape(OUT_ROWS, IDX_MINOR))
    out = (
        normed.reshape(B, C // 8, NUM_P // 128, 8, 128)
        .transpose(0, 2, 4, 1, 3)
        .reshape(B, NUM_P, C)
    )
    return out, patch_ids


# speedup vs baseline: 1.0185x; 1.0012x over previous
import functools

import jax
import jax.numpy as jnp
from jax import lax
from jax.experimental import pallas as pl
from jax.experimental.pallas import tpu as pltpu
from jax.experimental.pallas import tpu_sc as plsc

B = 4
C = 192
HW = 384 * 384
CHW = C * HW
NUM_P = 256
ELEMS = B * NUM_P * C

_info = plsc.get_sparse_core_info()
NC, NS, L = _info.num_cores, _info.num_subcores, _info.num_lanes
NW = NC * NS
ELEMS_PER_W = ELEMS // NW
IDX_MINOR = 128
IDX_MAJOR = ELEMS_PER_W // IDX_MINOR
OUT_ROWS = ELEMS // IDX_MINOR
C8_PER_W = 3
TILES_PER_B = 8


@functools.partial(
    pl.kernel,
    out_type=jax.ShapeDtypeStruct((ELEMS,), jnp.float32),
    mesh=plsc.VectorSubcoreMesh(core_axis_name="c", subcore_axis_name="s"),
    scratch_types=[
        pltpu.VMEM((NUM_P,), jnp.int32),
        pltpu.VMEM((ELEMS_PER_W,), jnp.int32),
        pltpu.VMEM((ELEMS_PER_W,), jnp.float32),
        pltpu.VMEM((2 * IDX_MINOR,), jnp.float32),
        pltpu.VMEM((TILES_PER_B * 2 * IDX_MINOR,), jnp.float32),
        pltpu.VMEM_SHARED((NS * 2 * IDX_MINOR,), jnp.float32),
        pltpu.SemaphoreType.DMA,
    ],
)
def _sc_gather(
    feats_hbm, pids_hbm, out_hbm,
    fpid_v, idx_v, rows_v, partial_v, group_v, shared_sq, sem,
):
    s = lax.axis_index("s")
    t = lax.axis_index("c") * NS + s
    b = lax.shift_right_logical(t, 3)
    for p128 in range(NUM_P // IDX_MINOR):
        pltpu.sync_copy(
            pids_hbm.at[pl.ds((p128 * B + b) * IDX_MINOR, IDX_MINOR)],
            fpid_v.at[pl.ds(p128 * IDX_MINOR, IDX_MINOR)],
        )

    def pid_fn(k, carry):
        p = fpid_v[pl.ds(k * L, L)]
        q = lax.shift_right_logical(p, 7)
        rem = lax.bitwise_and(p, 127)
        h = lax.shift_right_logical(q * 43691, 17)
        wq = q - 3 * h
        fpid_v[pl.ds(k * L, L)] = (
            lax.shift_right_logical(h, 3) * 3072
            + wq * 1024
            + lax.bitwise_and(h, 7) * 128
            + rem
        )
        return carry

    lax.fori_loop(0, NUM_P // L, pid_fn, 0)

    base_b = b * CHW

    def row_fn(j, carry):
        c8r = lax.shift_right_logical(j, 4)
        p128 = lax.bitwise_and(lax.shift_right_logical(j, 3), 1)
        cm8 = lax.bitwise_and(j, 7)
        c = ((lax.bitwise_and(t, 7) * C8_PER_W + c8r) * 8) + cm8
        base = base_b + c * HW
        for kk in range(IDX_MINOR // L):
            idx_v[pl.ds(j * IDX_MINOR + kk * L, L)] = (
                fpid_v[pl.ds(p128 * IDX_MINOR + kk * L, L)] + base
            )
        return carry

    lax.fori_loop(0, IDX_MAJOR, row_fn, 0)

    pltpu.async_copy(feats_hbm.at[idx_v], rows_v, sem).wait()

    NK = IDX_MINOR // L
    zeros = jnp.zeros((L,), jnp.float32)

    def sq_fn(i, accs):
        j0 = lax.shift_right_logical(i, 3) * 16 + lax.bitwise_and(i, 7)
        new = []
        for p128 in range(2):
            base = (j0 + p128 * 8) * IDX_MINOR
            for kk in range(NK):
                v = rows_v[pl.ds(base + kk * L, L)]
                new.append(accs[p128 * NK + kk] + v * v)
        return tuple(new)

    accs = lax.fori_loop(0, 24, sq_fn, tuple([zeros] * (2 * NK)))
    for q in range(2 * NK):
        partial_v[pl.ds(q * L, L)] = accs[q]

    pltpu.sync_copy(
        partial_v,
        shared_sq.at[pl.ds(pl.multiple_of(s * 2 * IDX_MINOR, 256), 2 * IDX_MINOR)],
    )
    plsc.subcore_barrier()
    g = lax.bitwise_and(s, 8) * 2 * IDX_MINOR
    pltpu.sync_copy(
        shared_sq.at[pl.ds(pl.multiple_of(g, 2048), TILES_PER_B * 2 * IDX_MINOR)],
        group_v,
    )

    invs = []
    for q in range(2 * NK):
        acc = group_v[pl.ds(q * L, L)]
        for i in range(1, TILES_PER_B):
            acc = acc + group_v[pl.ds(i * 2 * IDX_MINOR + q * L, L)]
        r = lax.bitcast_convert_type(
            0x5F3759DF
            - lax.shift_right_logical(lax.bitcast_convert_type(acc, jnp.int32), 1),
            jnp.float32,
        )
        half = acc * 0.5
        r = r * (1.5 - half * r * r)
        r = r * (1.5 - half * r * r)
        r = r * (1.5 - half * r * r)
        norm = acc * r
        invs.append(1.0 / (norm + 1e-7))

    def mul_fn(i, carry):
        j0 = lax.shift_right_logical(i, 3) * 16 + lax.bitwise_and(i, 7)
        for p128 in range(2):
            base = (j0 + p128 * 8) * IDX_MINOR
            for kk in range(NK):
                rows_v[pl.ds(base + kk * L, L)] = (
                    rows_v[pl.ds(base + kk * L, L)] * invs[p128 * NK + kk]
                )
        return carry

    lax.fori_loop(0, 24, mul_fn, 0)
    pltpu.sync_copy(
        rows_v,
        out_hbm.at[pl.ds(pl.multiple_of(t * ELEMS_PER_W, ELEMS_PER_W), ELEMS_PER_W)],
    )


def kernel(feats, num_patches, patch_ids):
    del num_patches
    feats_flat = (
        feats.reshape(B, C, 48, 8, 3, 128)
        .transpose(0, 1, 2, 4, 3, 5)
        .reshape(-1)
    )
    pids_flat = (
        patch_ids.reshape(B, NUM_P // 128, 128)
        .transpose(1, 0, 2)
        .reshape(-1)
    )
    normed = _sc_gather(feats_flat, pids_flat)
    out = (
        normed.reshape(B, C // 8, NUM_P // 128, 8, 128)
        .transpose(0, 2, 4, 1, 3)
        .reshape(B, NUM_P, C)
    )
    return out, patch_ids


# speedup vs baseline: 1.0354x; 1.0166x over previous
import functools

import jax
import jax.numpy as jnp
from jax import lax
from jax.experimental import pallas as pl
from jax.experimental.pallas import tpu as pltpu
from jax.experimental.pallas import tpu_sc as plsc

B = 4
C = 192
HW = 384 * 384
CHW = C * HW
NUM_P = 256
ELEMS = B * NUM_P * C

_info = plsc.get_sparse_core_info()
NC, NS, L = _info.num_cores, _info.num_subcores, _info.num_lanes
NW = NC * NS
ELEMS_PER_W = ELEMS // NW
IDX_MINOR = 128
IDX_MAJOR = ELEMS_PER_W // IDX_MINOR
OUT_ROWS = ELEMS // IDX_MINOR
C8_PER_W = 3
TILES_PER_B = 8


@functools.partial(
    pl.kernel,
    out_type=jax.ShapeDtypeStruct((ELEMS,), jnp.float32),
    mesh=plsc.VectorSubcoreMesh(core_axis_name="c", subcore_axis_name="s"),
    scratch_types=[
        pltpu.VMEM((NUM_P,), jnp.int32),
        pltpu.VMEM((ELEMS_PER_W,), jnp.int32),
        pltpu.VMEM((ELEMS_PER_W,), jnp.float32),
        pltpu.VMEM((2 * IDX_MINOR,), jnp.float32),
        pltpu.VMEM((TILES_PER_B * 2 * IDX_MINOR,), jnp.float32),
        pltpu.VMEM_SHARED((NS * 2 * IDX_MINOR,), jnp.float32),
        pltpu.SemaphoreType.DMA,
        pltpu.SemaphoreType.DMA,
        pltpu.SemaphoreType.DMA,
    ],
)
def _sc_gather(
    feats_hbm, pids_hbm, out_hbm,
    fpid_v, idx_v, rows_v, partial_v, group_v, shared_sq, sem0, sem1, sem2,
):
    sems = (sem0, sem1, sem2)
    s = lax.axis_index("s")
    t = lax.axis_index("c") * NS + s
    b = lax.shift_right_logical(t, 3)
    for p128 in range(NUM_P // IDX_MINOR):
        pltpu.sync_copy(
            pids_hbm.at[pl.ds((p128 * B + b) * IDX_MINOR, IDX_MINOR)],
            fpid_v.at[pl.ds(p128 * IDX_MINOR, IDX_MINOR)],
        )

    def pid_fn(k, carry):
        p = fpid_v[pl.ds(k * L, L)]
        q = lax.shift_right_logical(p, 7)
        rem = lax.bitwise_and(p, 127)
        h = lax.shift_right_logical(q * 43691, 17)
        wq = q - 3 * h
        fpid_v[pl.ds(k * L, L)] = (
            lax.shift_right_logical(h, 3) * 3072
            + wq * 1024
            + lax.bitwise_and(h, 7) * 128
            + rem
        )
        return carry

    lax.fori_loop(0, NUM_P // L, pid_fn, 0)

    base_b = b * CHW

    def row_fn(j, carry):
        c8r = lax.shift_right_logical(j, 4)
        p128 = lax.bitwise_and(lax.shift_right_logical(j, 3), 1)
        cm8 = lax.bitwise_and(j, 7)
        c = ((lax.bitwise_and(t, 7) * C8_PER_W + c8r) * 8) + cm8
        base = base_b + c * HW
        for kk in range(IDX_MINOR // L):
            idx_v[pl.ds(j * IDX_MINOR + kk * L, L)] = (
                fpid_v[pl.ds(p128 * IDX_MINOR + kk * L, L)] + base
            )
        return carry

    lax.fori_loop(0, IDX_MAJOR, row_fn, 0)

    SUB = ELEMS_PER_W // C8_PER_W
    for c8r in range(C8_PER_W):
        pltpu.async_copy(
            feats_hbm.at[idx_v.at[pl.ds(c8r * SUB, SUB)]],
            rows_v.at[pl.ds(c8r * SUB, SUB)],
            sems[c8r],
        )

    NK = IDX_MINOR // L
    zeros = jnp.zeros((L,), jnp.float32)
    accs = tuple([zeros] * (2 * NK))
    for c8r in range(C8_PER_W):
        pltpu.make_async_copy(
            out_hbm.at[pl.ds(pl.multiple_of(t * ELEMS_PER_W + c8r * SUB, SUB), SUB)],
            rows_v.at[pl.ds(c8r * SUB, SUB)],
            sems[c8r],
        ).wait()

        def sq_fn(i, a, _c8r=c8r):
            j0 = _c8r * 16 + i
            new = []
            for p128 in range(2):
                base = (j0 + p128 * 8) * IDX_MINOR
                for kk in range(NK):
                    v = rows_v[pl.ds(base + kk * L, L)]
                    new.append(a[p128 * NK + kk] + v * v)
            return tuple(new)

        accs = lax.fori_loop(0, 8, sq_fn, accs)
    for q in range(2 * NK):
        partial_v[pl.ds(q * L, L)] = accs[q]

    pltpu.sync_copy(
        partial_v,
        shared_sq.at[pl.ds(pl.multiple_of(s * 2 * IDX_MINOR, 256), 2 * IDX_MINOR)],
    )
    plsc.subcore_barrier()
    g = lax.bitwise_and(s, 8) * 2 * IDX_MINOR
    pltpu.sync_copy(
        shared_sq.at[pl.ds(pl.multiple_of(g, 2048), TILES_PER_B * 2 * IDX_MINOR)],
        group_v,
    )

    invs = []
    for q in range(2 * NK):
        acc = group_v[pl.ds(q * L, L)]
        for i in range(1, TILES_PER_B):
            acc = acc + group_v[pl.ds(i * 2 * IDX_MINOR + q * L, L)]
        r = lax.bitcast_convert_type(
            0x5F3759DF
            - lax.shift_right_logical(lax.bitcast_convert_type(acc, jnp.int32), 1),
            jnp.float32,
        )
        half = acc * 0.5
        r = r * (1.5 - half * r * r)
        r = r * (1.5 - half * r * r)
        r = r * (1.5 - half * r * r)
        norm = acc * r
        invs.append(1.0 / (norm + 1e-7))

    def mul_fn(i, carry):
        j0 = lax.shift_right_logical(i, 3) * 16 + lax.bitwise_and(i, 7)
        for p128 in range(2):
            base = (j0 + p128 * 8) * IDX_MINOR
            for kk in range(NK):
                rows_v[pl.ds(base + kk * L, L)] = (
                    rows_v[pl.ds(base + kk * L, L)] * invs[p128 * NK + kk]
                )
        return carry

    lax.fori_loop(0, 24, mul_fn, 0)
    pltpu.sync_copy(
        rows_v,
        out_hbm.at[pl.ds(pl.multiple_of(t * ELEMS_PER_W, ELEMS_PER_W), ELEMS_PER_W)],
    )


def kernel(feats, num_patches, patch_ids):
    del num_patches
    feats_flat = (
        feats.reshape(B, C, 48, 8, 3, 128)
        .transpose(0, 1, 2, 4, 3, 5)
        .reshape(-1)
    )
    pids_flat = (
        patch_ids.reshape(B, NUM_P // 128, 128)
        .transpose(1, 0, 2)
        .reshape(-1)
    )
    normed = _sc_gather(feats_flat, pids_flat)
    out = (
        normed.reshape(B, C // 8, NUM_P // 128, 8, 128)
        .transpose(0, 2, 4, 1, 3)
        .reshape(B, NUM_P, C)
    )
    return out, patch_ids


# speedup vs baseline: 1.0598x; 1.0236x over previous
import functools

import jax
import jax.numpy as jnp
from jax import lax
from jax.experimental import pallas as pl
from jax.experimental.pallas import tpu as pltpu
from jax.experimental.pallas import tpu_sc as plsc

B = 4
C = 192
HW = 384 * 384
CHW = C * HW
NUM_P = 256
ELEMS = B * NUM_P * C

_info = plsc.get_sparse_core_info()
NC, NS, L = _info.num_cores, _info.num_subcores, _info.num_lanes
NW = NC * NS
ELEMS_PER_W = ELEMS // NW
IDX_MINOR = 128
IDX_MAJOR = ELEMS_PER_W // IDX_MINOR
OUT_ROWS = ELEMS // IDX_MINOR
C8_PER_W = 3
TILES_PER_B = 8


@functools.partial(
    pl.kernel,
    out_type=jax.ShapeDtypeStruct((ELEMS,), jnp.float32),
    mesh=plsc.VectorSubcoreMesh(core_axis_name="c", subcore_axis_name="s"),
    scratch_types=[
        pltpu.VMEM((NUM_P,), jnp.int32),
        pltpu.VMEM((ELEMS_PER_W,), jnp.int32),
        pltpu.VMEM((ELEMS_PER_W,), jnp.float32),
        pltpu.VMEM((2 * IDX_MINOR,), jnp.float32),
        pltpu.VMEM((TILES_PER_B * 2 * IDX_MINOR,), jnp.float32),
        pltpu.VMEM_SHARED((NS * 2 * IDX_MINOR,), jnp.float32),
        pltpu.SemaphoreType.DMA,
        pltpu.SemaphoreType.DMA,
        pltpu.SemaphoreType.DMA,
    ],
)
def _sc_gather(
    feats_hbm, pids_hbm, out_hbm,
    fpid_v, idx_v, rows_v, partial_v, group_v, shared_sq, sem0, sem1, sem2,
):
    sems = (sem0, sem1, sem2)
    s = lax.axis_index("s")
    t = lax.axis_index("c") * NS + s
    b = lax.shift_right_logical(t, 3)
    for p128 in range(NUM_P // IDX_MINOR):
        pltpu.async_copy(
            pids_hbm.at[pl.ds((p128 * B + b) * IDX_MINOR, IDX_MINOR)],
            fpid_v.at[pl.ds(p128 * IDX_MINOR, IDX_MINOR)],
            sems[p128],
        )
    for p128 in range(NUM_P // IDX_MINOR):
        pltpu.make_async_copy(
            pids_hbm.at[pl.ds(p128 * IDX_MINOR, IDX_MINOR)],
            fpid_v.at[pl.ds(p128 * IDX_MINOR, IDX_MINOR)],
            sems[p128],
        ).wait()

    def pid_fn(k, carry):
        p = fpid_v[pl.ds(k * L, L)]
        q = lax.shift_right_logical(p, 7)
        rem = lax.bitwise_and(p, 127)
        h = lax.shift_right_logical(q * 43691, 17)
        wq = q - 3 * h
        fpid_v[pl.ds(k * L, L)] = (
            lax.shift_right_logical(h, 3) * 3072
            + wq * 1024
            + lax.bitwise_and(h, 7) * 128
            + rem
        )
        return carry

    lax.fori_loop(0, NUM_P // L, pid_fn, 0)

    base_b = b * CHW

    def row_fn(j, carry):
        c8r = lax.shift_right_logical(j, 4)
        p128 = lax.bitwise_and(lax.shift_right_logical(j, 3), 1)
        cm8 = lax.bitwise_and(j, 7)
        c = ((lax.bitwise_and(t, 7) * C8_PER_W + c8r) * 8) + cm8
        base = base_b + c * HW
        for kk in range(IDX_MINOR // L):
            idx_v[pl.ds(j * IDX_MINOR + kk * L, L)] = (
                fpid_v[pl.ds(p128 * IDX_MINOR + kk * L, L)] + base
            )
        return carry

    SUB = ELEMS_PER_W // C8_PER_W
    for c8r in range(C8_PER_W):
        lax.fori_loop(c8r * 16, (c8r + 1) * 16, row_fn, 0)
        pltpu.async_copy(
            feats_hbm.at[idx_v.at[pl.ds(c8r * SUB, SUB)]],
            rows_v.at[pl.ds(c8r * SUB, SUB)],
            sems[c8r],
        )

    NK = IDX_MINOR // L
    zeros = jnp.zeros((L,), jnp.float32)
    accs = tuple([zeros] * (2 * NK))
    for c8r in range(C8_PER_W):
        pltpu.make_async_copy(
            out_hbm.at[pl.ds(pl.multiple_of(t * ELEMS_PER_W + c8r * SUB, SUB), SUB)],
            rows_v.at[pl.ds(c8r * SUB, SUB)],
            sems[c8r],
        ).wait()

        def sq_fn(i, a, _c8r=c8r):
            j0 = _c8r * 16 + i
            new = []
            for p128 in range(2):
                base = (j0 + p128 * 8) * IDX_MINOR
                for kk in range(NK):
                    v = rows_v[pl.ds(base + kk * L, L)]
                    new.append(a[p128 * NK + kk] + v * v)
            return tuple(new)

        accs = lax.fori_loop(0, 8, sq_fn, accs)
    for q in range(2 * NK):
        partial_v[pl.ds(q * L, L)] = accs[q]

    pltpu.sync_copy(
        partial_v,
        shared_sq.at[pl.ds(pl.multiple_of(s * 2 * IDX_MINOR, 256), 2 * IDX_MINOR)],
    )
    plsc.subcore_barrier()
    g = lax.bitwise_and(s, 8) * 2 * IDX_MINOR
    pltpu.sync_copy(
        shared_sq.at[pl.ds(pl.multiple_of(g, 2048), TILES_PER_B * 2 * IDX_MINOR)],
        group_v,
    )

    invs = []
    for q in range(2 * NK):
        acc = group_v[pl.ds(q * L, L)]
        for i in range(1, TILES_PER_B):
            acc = acc + group_v[pl.ds(i * 2 * IDX_MINOR + q * L, L)]
        r = lax.bitcast_convert_type(
            0x5F3759DF
            - lax.shift_right_logical(lax.bitcast_convert_type(acc, jnp.int32), 1),
            jnp.float32,
        )
        half = acc * 0.5
        r = r * (1.5 - half * r * r)
        r = r * (1.5 - half * r * r)
        r = r * (1.5 - half * r * r)
        norm = acc * r
        invs.append(1.0 / (norm + 1e-7))

    def mul_fn(i, carry):
        j0 = lax.shift_right_logical(i, 3) * 16 + lax.bitwise_and(i, 7)
        for p128 in range(2):
            base = (j0 + p128 * 8) * IDX_MINOR
            for kk in range(NK):
                rows_v[pl.ds(base + kk * L, L)] = (
                    rows_v[pl.ds(base + kk * L, L)] * invs[p128 * NK + kk]
                )
        return carry

    lax.fori_loop(0, 24, mul_fn, 0)
    pltpu.sync_copy(
        rows_v,
        out_hbm.at[pl.ds(pl.multiple_of(t * ELEMS_PER_W, ELEMS_PER_W), ELEMS_PER_W)],
    )


def kernel(feats, num_patches, patch_ids):
    del num_patches
    feats_flat = (
        feats.reshape(B, C, 48, 8, 3, 128)
        .transpose(0, 1, 2, 4, 3, 5)
        .reshape(-1)
    )
    pids_flat = (
        patch_ids.reshape(B, NUM_P // 128, 128)
        .transpose(1, 0, 2)
        .reshape(-1)
    )
    normed = _sc_gather(feats_flat, pids_flat)
    out = (
        normed.reshape(B, C // 8, NUM_P // 128, 8, 128)
        .transpose(0, 2, 4, 1, 3)
        .reshape(B, NUM_P, C)
    )
    return out, patch_ids


# speedup vs baseline: 1.0617x; 1.0018x over previous
import functools

import jax
import jax.numpy as jnp
from jax import lax
from jax.experimental import pallas as pl
from jax.experimental.pallas import tpu as pltpu
from jax.experimental.pallas import tpu_sc as plsc

B = 4
C = 192
HW = 384 * 384
CHW = C * HW
NUM_P = 256
ELEMS = B * NUM_P * C

_info = plsc.get_sparse_core_info()
NC, NS, L = _info.num_cores, _info.num_subcores, _info.num_lanes
NW = NC * NS
ELEMS_PER_W = ELEMS // NW
IDX_MINOR = 128
C8_PER_W = 3
TILES_PER_B = 8


@functools.partial(
    pl.kernel,
    out_type=jax.ShapeDtypeStruct((ELEMS,), jnp.float32),
    mesh=plsc.VectorSubcoreMesh(core_axis_name="c", subcore_axis_name="s"),
    scratch_types=[
        pltpu.VMEM((NUM_P,), jnp.int32),
        pltpu.VMEM((ELEMS_PER_W,), jnp.int32),
        pltpu.VMEM((ELEMS_PER_W,), jnp.float32),
        pltpu.VMEM((2 * IDX_MINOR,), jnp.float32),
        pltpu.VMEM((TILES_PER_B * 2 * IDX_MINOR,), jnp.float32),
        pltpu.VMEM_SHARED((NS * 2 * IDX_MINOR,), jnp.float32),
        pltpu.SemaphoreType.DMA,
        pltpu.SemaphoreType.DMA,
        pltpu.SemaphoreType.DMA,
    ],
)
def _sc_gather(
    feats_hbm, pids_hbm, out_hbm,
    fpid_v, idx_v, rows_v, partial_v, group_v, shared_sq, sem0, sem1, sem2,
):
    sems = (sem0, sem1, sem2)
    s = lax.axis_index("s")
    t = lax.axis_index("c") * NS + s
    b = lax.shift_right_logical(t, 3)
    for p128 in range(NUM_P // IDX_MINOR):
        pltpu.async_copy(
            pids_hbm.at[pl.ds((p128 * B + b) * IDX_MINOR, IDX_MINOR)],
            fpid_v.at[pl.ds(p128 * IDX_MINOR, IDX_MINOR)],
            sems[p128],
        )
    for p128 in range(NUM_P // IDX_MINOR):
        pltpu.make_async_copy(
            pids_hbm.at[pl.ds(p128 * IDX_MINOR, IDX_MINOR)],
            fpid_v.at[pl.ds(p128 * IDX_MINOR, IDX_MINOR)],
            sems[p128],
        ).wait()

    def pid_fn(k, carry):
        p = fpid_v[pl.ds(k * L, L)]
        q = lax.shift_right_logical(p, 7)
        rem = lax.bitwise_and(p, 127)
        h = lax.shift_right_logical(q * 43691, 17)
        wq = q - 3 * h
        fpid_v[pl.ds(k * L, L)] = (
            lax.shift_right_logical(h, 3) * 3072
            + wq * 1024
            + lax.bitwise_and(h, 7) * 128
            + rem
        )
        return carry

    lax.fori_loop(0, NUM_P // L, pid_fn, 0)

    base_b = b * CHW

    def row_fn(j, carry):
        c8r = lax.shift_right_logical(j, 4)
        p128 = lax.bitwise_and(lax.shift_right_logical(j, 3), 1)
        cm8 = lax.bitwise_and(j, 7)
        c = ((lax.bitwise_and(t, 7) * C8_PER_W + c8r) * 8) + cm8
        base = base_b + c * HW
        for kk in range(IDX_MINOR // L):
            idx_v[pl.ds(j * IDX_MINOR + kk * L, L)] = (
                fpid_v[pl.ds(p128 * IDX_MINOR + kk * L, L)] + base
            )
        return carry

    SUB = ELEMS_PER_W // C8_PER_W
    for c8r in range(C8_PER_W):
        lax.fori_loop(c8r * 16, (c8r + 1) * 16, row_fn, 0)
        pltpu.async_copy(
            feats_hbm.at[idx_v.at[pl.ds(c8r * SUB, SUB)]],
            rows_v.at[pl.ds(c8r * SUB, SUB)],
            sems[c8r],
        )

    NK = IDX_MINOR // L
    zeros = jnp.zeros((L,), jnp.float32)
    accs = tuple([zeros] * (2 * NK))
    for c8r in range(C8_PER_W):
        pltpu.make_async_copy(
            out_hbm.at[pl.ds(pl.multiple_of(t * ELEMS_PER_W + c8r * SUB, SUB), SUB)],
            rows_v.at[pl.ds(c8r * SUB, SUB)],
            sems[c8r],
        ).wait()

        def sq_fn(i, a, _c8r=c8r):
            j0 = _c8r * 16 + i
            new = []
            for p128 in range(2):
                base = (j0 + p128 * 8) * IDX_MINOR
                for kk in range(NK):
                    v = rows_v[pl.ds(base + kk * L, L)]
                    new.append(a[p128 * NK + kk] + v * v)
            return tuple(new)

        accs = lax.fori_loop(0, 8, sq_fn, accs)
    for q in range(2 * NK):
        partial_v[pl.ds(q * L, L)] = accs[q]

    pltpu.sync_copy(
        partial_v,
        shared_sq.at[pl.ds(pl.multiple_of(s * 2 * IDX_MINOR, 256), 2 * IDX_MINOR)],
    )
    plsc.subcore_barrier()
    g = lax.bitwise_and(s, 8) * 2 * IDX_MINOR
    pltpu.sync_copy(
        shared_sq.at[pl.ds(pl.multiple_of(g, 2048), TILES_PER_B * 2 * IDX_MINOR)],
        group_v,
    )

    invs = []
    for q in range(2 * NK):
        acc = group_v[pl.ds(q * L, L)]
        for i in range(1, TILES_PER_B):
            acc = acc + group_v[pl.ds(i * 2 * IDX_MINOR + q * L, L)]
        r = lax.bitcast_convert_type(
            0x5F3759DF
            - lax.shift_right_logical(lax.bitcast_convert_type(acc, jnp.int32), 1),
            jnp.float32,
        )
        half = acc * 0.5
        r = r * (1.5 - half * r * r)
        r = r * (1.5 - half * r * r)
        r = r * (1.5 - half * r * r)
        norm = acc * r
        invs.append(1.0 / (norm + 1e-7))

    def mul_fn(i, carry):
        j0 = lax.shift_right_logical(i, 3) * 16 + lax.bitwise_and(i, 7)
        for p128 in range(2):
            base = (j0 + p128 * 8) * IDX_MINOR
            for kk in range(NK):
                rows_v[pl.ds(base + kk * L, L)] = (
                    rows_v[pl.ds(base + kk * L, L)] * invs[p128 * NK + kk]
                )
        return carry

    lax.fori_loop(0, 24, mul_fn, 0)
    pltpu.sync_copy(
        rows_v,
        out_hbm.at[pl.ds(pl.multiple_of(t * ELEMS_PER_W, ELEMS_PER_W), ELEMS_PER_W)],
    )


def kernel(feats, num_patches, patch_ids):
    del num_patches
    feats_flat = (
        feats.reshape(B, C, 48, 8, 3, 128)
        .transpose(0, 1, 2, 4, 3, 5)
        .reshape(-1)
    )
    pids_flat = (
        patch_ids.reshape(B, NUM_P // 128, 128)
        .transpose(1, 0, 2)
        .reshape(-1)
    )
    normed = _sc_gather(feats_flat, pids_flat)
    out = (
        normed.reshape(B, C // 8, NUM_P // 128, 8, 128)
        .transpose(0, 2, 4, 1, 3)
        .reshape(B, NUM_P, C)
    )
    return out, patch_ids
